# Initial kernel scaffold; baseline (speedup 1.0000x reference)
import jax
import jax.numpy as jnp
from jax.experimental import pallas as pl


def kernel(features, conv_w, conv_b, cls_w, cls_b, reg_w, reg_b, anchor0):
    raise NotImplementedError("write your pallas kernel here")


# speedup vs baseline: 19.8167x; 19.8167x over previous
import jax
import jax.numpy as jnp
from jax.experimental import pallas as pl

H = 80
W = 80
A = 3
C = 256
RW = 88
NMS_T = 0.7
MAX_OUT = 1000
N = H * W * A
LANES = 128
ROWS = 152
NPAD = ROWS * LANES
BLK = 512
POUT = 7168
PIN = 7344


def _conv_body(x_ref, w_ref, b_ref, wh_ref, hb_ref, o_ref):
    i = pl.program_id(0)
    base = i * BLK
    acc = jnp.zeros((BLK, C), jnp.float32)
    for dy in range(3):
        for dx in range(3):
            acc = acc + jnp.dot(
                x_ref[dx, pl.ds(base + dy * RW, BLK), :], w_ref[dy * 3 + dx],
                preferred_element_type=jnp.float32)
    x = jnp.maximum(acc + b_ref[...], 0.0)
    o_ref[...] = jnp.dot(x, wh_ref[...],
                         preferred_element_type=jnp.float32) + hb_ref[...]


def _nms_body(sc_ref, dx_ref, dy_ref, dw_ref, dh_ref,
              ax1_ref, ay1_ref, ax2_ref, ay2_ref, o_ref):
    scores = sc_ref[...]
    aw = ax2_ref[...] - ax1_ref[...]
    ah = ay2_ref[...] - ay1_ref[...]
    acx = ax1_ref[...] + 0.5 * aw
    acy = ay1_ref[...] + 0.5 * ah
    dw = jnp.minimum(dw_ref[...], 4.0)
    dh = jnp.minimum(dh_ref[...], 4.0)
    pcx = dx_ref[...] * aw + acx
    pcy = dy_ref[...] * ah + acy
    pw = jnp.exp(dw) * aw
    ph = jnp.exp(dh) * ah
    x1 = pcx - 0.5 * pw
    y1 = pcy - 0.5 * ph
    x2 = pcx + 0.5 * pw
    y2 = pcy + 0.5 * ph
    areas = (x2 - x1) * (y2 - y1)
    ridx = jax.lax.broadcasted_iota(jnp.int32, (ROWS, LANES), 0)
    cidx = jax.lax.broadcasted_iota(jnp.int32, (ROWS, LANES), 1)
    flat = ridx * LANES + cidx
    lane4 = jax.lax.broadcasted_iota(jnp.int32, (1, LANES), 1)
    masked0 = jnp.where(flat < N, scores, -1.0)

    def body(i, masked):
        m = jnp.max(masked)
        any_valid = m > -0.5
        is_max = masked == m
        idxsel = jnp.min(jnp.where(is_max, flat, jnp.int32(2**30)))
        onehot = flat == idxsel
        bx1 = jnp.sum(jnp.where(onehot, x1, 0.0))
        by1 = jnp.sum(jnp.where(onehot, y1, 0.0))
        bx2 = jnp.sum(jnp.where(onehot, x2, 0.0))
        by2 = jnp.sum(jnp.where(onehot, y2, 0.0))
        asel = (bx2 - bx1) * (by2 - by1)
        xx1 = jnp.maximum(bx1, x1)
        yy1 = jnp.maximum(by1, y1)
        xx2 = jnp.minimum(bx2, x2)
        yy2 = jnp.minimum(by2, y2)
        inter = jnp.maximum(xx2 - xx1, 0.0) * jnp.maximum(yy2 - yy1, 0.0)
        iou = inter / (areas + asel - inter + 1e-9)
        keep = jnp.logical_and(iou <= NMS_T, jnp.logical_not(onehot))
        new_masked = jnp.where(keep, masked, -1.0)
        masked = jnp.where(any_valid, new_masked, masked)
        row = jnp.where(lane4 == 0, bx1,
              jnp.where(lane4 == 1, by1,
              jnp.where(lane4 == 2, bx2,
              jnp.where(lane4 == 3, by2, 0.0))))
        row = jnp.where(any_valid, row, -1.0)
        o_ref[pl.ds(i, 1), :] = row
        return masked

    jax.lax.fori_loop(0, MAX_OUT, body, masked0)


def kernel(features, conv_w, conv_b, cls_w, cls_b, reg_w, reg_b, anchor0):
    xin = jnp.transpose(features[0], (1, 2, 0))
    xpad = jnp.pad(xin, ((1, 1), (1, RW - 1 - W), (0, 0)))
    xbig = jnp.pad(xpad.reshape(82 * RW, C), ((0, PIN + 2 - 82 * RW), (0, 0)))
    xflat = jnp.stack(
        [xbig[0:PIN], xbig[1:PIN + 1], xbig[2:PIN + 2]])
    w9 = jnp.transpose(conv_w, (2, 3, 1, 0)).reshape(9, C, C)
    whead = jnp.concatenate([cls_w[:, :, 0, 0].T, reg_w[:, :, 0, 0].T], axis=1)
    whead = jnp.pad(whead, ((0, 0), (0, LANES - 15)))
    hbias = jnp.pad(jnp.concatenate([cls_b, reg_b]),
                    (0, LANES - 15)).reshape(1, LANES)
    bias = conv_b.reshape(1, C)

    heads = pl.pallas_call(
        _conv_body,
        grid=(POUT // BLK,),
        in_specs=[
            pl.BlockSpec((3, PIN, C), lambda i: (0, 0, 0)),
            pl.BlockSpec((9, C, C), lambda i: (0, 0, 0)),
            pl.BlockSpec((1, C), lambda i: (0, 0)),
            pl.BlockSpec((C, LANES), lambda i: (0, 0)),
            pl.BlockSpec((1, LANES), lambda i: (0, 0)),
        ],
        out_specs=pl.BlockSpec((BLK, LANES), lambda i: (i, 0)),
        out_shape=jax.ShapeDtypeStruct((POUT, LANES), jnp.float32),
    )(xflat, w9, bias, whead, hbias)

    hb = heads[:H * RW].reshape(H, RW, LANES)[:, :W, :]
    logits = hb[:, :, :3].reshape(N)
    scores = jax.nn.sigmoid(logits)
    deltas = hb[:, :, 3:15].reshape(H, W, A, 4).reshape(N, 4)
    anch = anchor0.reshape(N, 4)

    def to_grid(v):
        return jnp.pad(v, (0, NPAD - N)).reshape(ROWS, LANES)

    args = [to_grid(scores)]
    for t in range(4):
        args.append(to_grid(deltas[:, t]))
    for t in range(4):
        args.append(to_grid(anch[:, t]))

    out = pl.pallas_call(
        _nms_body,
        out_shape=jax.ShapeDtypeStruct((1008, LANES), jnp.float32),
    )(*args)
    return out[:MAX_OUT, :4].reshape(1, MAX_OUT, 4)


# speedup vs baseline: 20.3475x; 1.0268x over previous
import jax
import jax.numpy as jnp
from jax.experimental import pallas as pl
from jax.experimental.pallas import tpu as pltpu

H = 80
W = 80
A = 3
C = 256
RW = 88
NMS_T = 0.7
MAX_OUT = 1000
N = H * W * A
LANES = 128
ROWS = 152
NPAD = ROWS * LANES
BLK = 512
POUT = 7168
PIN = 7344


def _conv_body(x_ref, w_ref, b_ref, wh_ref, hb_ref, o_ref):
    i = pl.program_id(0)
    base = i * BLK
    acc = jnp.zeros((BLK, C), jnp.float32)
    for dy in range(3):
        for dx in range(3):
            acc = acc + jnp.dot(
                x_ref[dx, pl.ds(base + dy * RW, BLK), :], w_ref[dy * 3 + dx],
                preferred_element_type=jnp.float32)
    x = jnp.maximum(acc + b_ref[...], 0.0)
    o_ref[...] = jnp.dot(x, wh_ref[...],
                         preferred_element_type=jnp.float32) + hb_ref[...]


def _nms_body(sc_ref, dx_ref, dy_ref, dw_ref, dh_ref,
              ax1_ref, ay1_ref, ax2_ref, ay2_ref, o_ref,
              cx1_ref, cy1_ref, cx2_ref, cy2_ref):
    scores = sc_ref[...]
    aw = ax2_ref[...] - ax1_ref[...]
    ah = ay2_ref[...] - ay1_ref[...]
    acx = ax1_ref[...] + 0.5 * aw
    acy = ay1_ref[...] + 0.5 * ah
    dw = jnp.minimum(dw_ref[...], 4.0)
    dh = jnp.minimum(dh_ref[...], 4.0)
    pcx = dx_ref[...] * aw + acx
    pcy = dy_ref[...] * ah + acy
    pw = jnp.exp(dw) * aw
    ph = jnp.exp(dh) * ah
    x1 = pcx - 0.5 * pw
    y1 = pcy - 0.5 * ph
    x2 = pcx + 0.5 * pw
    y2 = pcy + 0.5 * ph
    areas = (x2 - x1) * (y2 - y1)
    cx1_ref[...] = x1.reshape(ROWS, 1, LANES)
    cy1_ref[...] = y1.reshape(ROWS, 1, LANES)
    cx2_ref[...] = x2.reshape(ROWS, 1, LANES)
    cy2_ref[...] = y2.reshape(ROWS, 1, LANES)
    ridx = jax.lax.broadcasted_iota(jnp.int32, (ROWS, LANES), 0)
    cidx = jax.lax.broadcasted_iota(jnp.int32, (ROWS, LANES), 1)
    flat = ridx * LANES + cidx
    lanei = jax.lax.broadcasted_iota(jnp.int32, (1, LANES), 1)
    masked0 = jnp.where(flat < N, scores, -1.0)

    def body(i, masked):
        m = jnp.max(masked)
        any_valid = m > -0.5
        is_max = masked == m
        idxsel = jnp.min(jnp.where(is_max, flat, jnp.int32(2**30)))
        safe = jnp.where(any_valid, idxsel, 0)
        r = jax.lax.shift_right_logical(safe, 7)
        c = jax.lax.bitwise_and(safe, 127)
        lsel = lanei == c
        bx1 = jnp.sum(jnp.where(lsel, cx1_ref[pl.ds(r, 1)].reshape(1, LANES), 0.0))
        by1 = jnp.sum(jnp.where(lsel, cy1_ref[pl.ds(r, 1)].reshape(1, LANES), 0.0))
        bx2 = jnp.sum(jnp.where(lsel, cx2_ref[pl.ds(r, 1)].reshape(1, LANES), 0.0))
        by2 = jnp.sum(jnp.where(lsel, cy2_ref[pl.ds(r, 1)].reshape(1, LANES), 0.0))
        asel = (bx2 - bx1) * (by2 - by1)
        xx1 = jnp.maximum(bx1, x1)
        yy1 = jnp.maximum(by1, y1)
        xx2 = jnp.minimum(bx2, x2)
        yy2 = jnp.minimum(by2, y2)
        inter = jnp.maximum(xx2 - xx1, 0.0) * jnp.maximum(yy2 - yy1, 0.0)
        iou = inter / (areas + asel - inter + 1e-9)
        keep = jnp.logical_or(
            jnp.logical_and(iou <= NMS_T, flat != idxsel),
            jnp.logical_not(any_valid))
        masked = jnp.where(keep, masked, -1.0)
        row = jnp.where(lanei == 0, bx1,
              jnp.where(lanei == 1, by1,
              jnp.where(lanei == 2, bx2,
              jnp.where(lanei == 3, by2, 0.0))))
        row = jnp.where(any_valid, row, -1.0)
        o_ref[pl.ds(i, 1), :] = row
        return masked

    jax.lax.fori_loop(0, MAX_OUT, body, masked0)


def kernel(features, conv_w, conv_b, cls_w, cls_b, reg_w, reg_b, anchor0):
    xin = jnp.transpose(features[0], (1, 2, 0))
    xpad = jnp.pad(xin, ((1, 1), (1, RW - 1 - W), (0, 0)))
    xbig = jnp.pad(xpad.reshape(82 * RW, C), ((0, PIN + 2 - 82 * RW), (0, 0)))
    xflat = jnp.stack(
        [xbig[0:PIN], xbig[1:PIN + 1], xbig[2:PIN + 2]])
    w9 = jnp.transpose(conv_w, (2, 3, 1, 0)).reshape(9, C, C)
    whead = jnp.concatenate([cls_w[:, :, 0, 0].T, reg_w[:, :, 0, 0].T], axis=1)
    whead = jnp.pad(whead, ((0, 0), (0, LANES - 15)))
    hbias = jnp.pad(jnp.concatenate([cls_b, reg_b]),
                    (0, LANES - 15)).reshape(1, LANES)
    bias = conv_b.reshape(1, C)

    heads = pl.pallas_call(
        _conv_body,
        grid=(POUT // BLK,),
        in_specs=[
            pl.BlockSpec((3, PIN, C), lambda i: (0, 0, 0)),
            pl.BlockSpec((9, C, C), lambda i: (0, 0, 0)),
            pl.BlockSpec((1, C), lambda i: (0, 0)),
            pl.BlockSpec((C, LANES), lambda i: (0, 0)),
            pl.BlockSpec((1, LANES), lambda i: (0, 0)),
        ],
        out_specs=pl.BlockSpec((BLK, LANES), lambda i: (i, 0)),
        out_shape=jax.ShapeDtypeStruct((POUT, LANES), jnp.float32),
    )(xflat, w9, bias, whead, hbias)

    hb = heads[:H * RW].reshape(H, RW, LANES)[:, :W, :]
    logits = hb[:, :, :3].reshape(N)
    scores = jax.nn.sigmoid(logits)
    deltas = hb[:, :, 3:15].reshape(H, W, A, 4).reshape(N, 4)
    anch = anchor0.reshape(N, 4)

    def to_grid(v):
        return jnp.pad(v, (0, NPAD - N)).reshape(ROWS, LANES)

    args = [to_grid(scores)]
    for t in range(4):
        args.append(to_grid(deltas[:, t]))
    for t in range(4):
        args.append(to_grid(anch[:, t]))

    out = pl.pallas_call(
        _nms_body,
        out_shape=jax.ShapeDtypeStruct((1008, LANES), jnp.float32),
        scratch_shapes=[pltpu.VMEM((ROWS, 1, LANES), jnp.float32)
                        for _ in range(4)],
    )(*args)
    return out[:MAX_OUT, :4].reshape(1, MAX_OUT, 4)
